# initial kernel scaffold (unmeasured)
import functools

import jax
import jax.numpy as jnp
from jax import lax
from jax.experimental import pallas as pl
from jax.experimental.pallas import tpu as pltpu

N_DEV = 4
N_HOPS = N_DEV - 1


def kernel(x, w_mat):
    m_per, k = x.shape
    _, n_per = w_mat.shape
    half = m_per // 2
    m_glob = N_DEV * m_per

    x_bf = x.astype(jnp.bfloat16)
    w_bf = w_mat.astype(jnp.bfloat16)

    def body(x_ref, w_ref, out_ref, cw_ref, ccw_ref, staging_ref,
             send_cw, recv_cw, send_ccw, recv_ccw, copy_sems):
        my = lax.axis_index("i")
        left = (my - 1) % N_DEV
        right = (my + 1) % N_DEV

        barrier_sem = pltpu.get_barrier_semaphore()
        for nbr in (left, right):
            pl.semaphore_signal(
                barrier_sem, inc=1,
                device_id=(nbr,), device_id_type=pl.DeviceIdType.MESH,
            )
        pl.semaphore_wait(barrier_sem, 2)

        def make_cw(h, src):
            return pltpu.make_async_remote_copy(
                src_ref=src,
                dst_ref=cw_ref.at[h],
                send_sem=send_cw.at[h],
                recv_sem=recv_cw.at[h],
                device_id=(right,),
                device_id_type=pl.DeviceIdType.MESH,
            )

        def make_ccw(h, src):
            return pltpu.make_async_remote_copy(
                src_ref=src,
                dst_ref=ccw_ref.at[h],
                send_sem=send_ccw.at[h],
                recv_sem=recv_ccw.at[h],
                device_id=(left,),
                device_id_type=pl.DeviceIdType.MESH,
            )

        cw = [None] * N_HOPS
        ccw = [None] * N_HOPS
        cw[0] = make_cw(0, x_ref.at[pl.ds(0, half)])
        ccw[0] = make_ccw(0, x_ref.at[pl.ds(half, half)])
        cw[0].start()
        ccw[0].start()

        copies = [None, None]
        ctr = [0]

        def compute_store(xv, row_start):
            slot = ctr[0] % 2
            if copies[slot] is not None:
                copies[slot].wait()
            staging_ref[slot] = jnp.maximum(
                jnp.dot(xv, w_ref[...], preferred_element_type=jnp.float32),
                0.0,
            )
            cp = pltpu.make_async_copy(
                staging_ref.at[slot],
                out_ref.at[pl.ds(row_start, half)],
                copy_sems.at[slot],
            )
            cp.start()
            copies[slot] = cp
            ctr[0] += 1

        compute_store(x_ref[pl.ds(0, half)], my * m_per)
        compute_store(x_ref[pl.ds(half, half)], my * m_per + half)

        for h in range(N_HOPS):
            cw[h].wait_recv()
            if h + 1 < N_HOPS:
                cw[h + 1] = make_cw(h + 1, cw_ref.at[h])
                cw[h + 1].start()
            ccw[h].wait_recv()
            if h + 1 < N_HOPS:
                ccw[h + 1] = make_ccw(h + 1, ccw_ref.at[h])
                ccw[h + 1].start()

            o_cw = (my - 1 - h) % N_DEV
            compute_store(cw_ref[h], o_cw * m_per)
            o_ccw = (my + 1 + h) % N_DEV
            compute_store(ccw_ref[h], o_ccw * m_per + half)

        for h in range(N_HOPS):
            cw[h].wait_send()
            ccw[h].wait_send()
        copies[0].wait()
        copies[1].wait()

    return pl.pallas_call(
        body,
        out_shape=jax.ShapeDtypeStruct((m_glob, n_per), jnp.float32),
        in_specs=[
            pl.BlockSpec(memory_space=pltpu.VMEM),
            pl.BlockSpec(memory_space=pltpu.VMEM),
        ],
        out_specs=pl.BlockSpec(memory_space=pltpu.ANY),
        scratch_shapes=[
            pltpu.VMEM((N_HOPS, half, k), jnp.bfloat16),
            pltpu.VMEM((N_HOPS, half, k), jnp.bfloat16),
            pltpu.VMEM((2, half, n_per), jnp.float32),
            pltpu.SemaphoreType.DMA((N_HOPS,)),
            pltpu.SemaphoreType.DMA((N_HOPS,)),
            pltpu.SemaphoreType.DMA((N_HOPS,)),
            pltpu.SemaphoreType.DMA((N_HOPS,)),
            pltpu.SemaphoreType.DMA((2,)),
        ],
        compiler_params=pltpu.CompilerParams(collective_id=0),
    )(x_bf, w_bf)


# baseline (device time: 217772 ns/iter reference)
import functools

import jax
import jax.numpy as jnp
from jax import lax
from jax.experimental import pallas as pl
from jax.experimental.pallas import tpu as pltpu

N_DEV = 4
N_HOPS = N_DEV - 1


def kernel(x, w_mat):
    m_per, k = x.shape
    _, n_per = w_mat.shape
    half = m_per // 2
    m_glob = N_DEV * m_per

    x_bf = x.astype(jnp.bfloat16)
    w_bf = w_mat.astype(jnp.bfloat16)

    def body(x_ref, w_ref, out_ref, cw_ref, ccw_ref, staging_ref,
             send_cw, recv_cw, send_ccw, recv_ccw, copy_sems,
             credit_cw, credit_ccw):
        my = lax.axis_index("i")
        left = (my - 1) % N_DEV
        right = (my + 1) % N_DEV

        barrier_sem = pltpu.get_barrier_semaphore()
        for nbr in (left, right):
            pl.semaphore_signal(
                barrier_sem, inc=1,
                device_id=(nbr,), device_id_type=pl.DeviceIdType.MESH,
            )
        pl.semaphore_wait(barrier_sem, 2)

        def make_cw(h, src):
            return pltpu.make_async_remote_copy(
                src_ref=src,
                dst_ref=cw_ref.at[h % 2],
                send_sem=send_cw.at[h],
                recv_sem=recv_cw.at[h],
                device_id=(right,),
                device_id_type=pl.DeviceIdType.MESH,
            )

        def make_ccw(h, src):
            return pltpu.make_async_remote_copy(
                src_ref=src,
                dst_ref=ccw_ref.at[h % 2],
                send_sem=send_ccw.at[h],
                recv_sem=recv_ccw.at[h],
                device_id=(left,),
                device_id_type=pl.DeviceIdType.MESH,
            )

        cw = [None] * N_HOPS
        ccw = [None] * N_HOPS
        cw[0] = make_cw(0, x_ref.at[pl.ds(0, half)])
        ccw[0] = make_ccw(0, x_ref.at[pl.ds(half, half)])
        cw[0].start()
        ccw[0].start()

        n_tile = 512
        copies = [None, None]
        ctr = [0]

        def compute_store(chunk_ref, row_start):
            xv = chunk_ref[...]
            for j in range(n_per // n_tile):
                slot = ctr[0] % 2
                if copies[slot] is not None:
                    copies[slot].wait()
                staging_ref[slot] = jnp.maximum(
                    jnp.dot(
                        xv,
                        w_ref[:, pl.ds(j * n_tile, n_tile)],
                        preferred_element_type=jnp.float32,
                    ),
                    0.0,
                )
                cp = pltpu.make_async_copy(
                    staging_ref.at[slot],
                    out_ref.at[pl.ds(row_start, half), pl.ds(j * n_tile, n_tile)],
                    copy_sems.at[slot],
                )
                cp.start()
                copies[slot] = cp
                ctr[0] += 1

        compute_store(x_ref.at[pl.ds(0, half)], my * m_per)
        compute_store(x_ref.at[pl.ds(half, half)], my * m_per + half)

        for h in range(N_HOPS):
            cw[h].wait_recv()
            ccw[h].wait_recv()
            if h + 1 < N_HOPS:
                if h + 1 == 2:
                    cw[1].wait_send()
                    ccw[1].wait_send()
                    pl.semaphore_signal(
                        credit_cw, inc=1,
                        device_id=(left,),
                        device_id_type=pl.DeviceIdType.MESH,
                    )
                    pl.semaphore_signal(
                        credit_ccw, inc=1,
                        device_id=(right,),
                        device_id_type=pl.DeviceIdType.MESH,
                    )
                    pl.semaphore_wait(credit_cw, 1)
                    pl.semaphore_wait(credit_ccw, 1)
                cw[h + 1] = make_cw(h + 1, cw_ref.at[h % 2])
                cw[h + 1].start()
                ccw[h + 1] = make_ccw(h + 1, ccw_ref.at[h % 2])
                ccw[h + 1].start()

            o_cw = (my - 1 - h) % N_DEV
            compute_store(cw_ref.at[h % 2], o_cw * m_per)
            o_ccw = (my + 1 + h) % N_DEV
            compute_store(ccw_ref.at[h % 2], o_ccw * m_per + half)

        for h in (0, 2):
            cw[h].wait_send()
            ccw[h].wait_send()
        copies[0].wait()
        copies[1].wait()

    return pl.pallas_call(
        body,
        out_shape=jax.ShapeDtypeStruct((m_glob, n_per), jnp.float32),
        in_specs=[
            pl.BlockSpec(memory_space=pltpu.MemorySpace.VMEM),
            pl.BlockSpec(memory_space=pltpu.MemorySpace.VMEM),
        ],
        out_specs=pl.BlockSpec(memory_space=pl.ANY),
        scratch_shapes=[
            pltpu.VMEM((2, half, k), jnp.bfloat16),
            pltpu.VMEM((2, half, k), jnp.bfloat16),
            pltpu.VMEM((2, half, 512), jnp.float32),
            pltpu.SemaphoreType.DMA((N_HOPS,)),
            pltpu.SemaphoreType.DMA((N_HOPS,)),
            pltpu.SemaphoreType.DMA((N_HOPS,)),
            pltpu.SemaphoreType.DMA((N_HOPS,)),
            pltpu.SemaphoreType.DMA((2,)),
            pltpu.SemaphoreType.REGULAR,
            pltpu.SemaphoreType.REGULAR,
        ],
        compiler_params=pltpu.CompilerParams(
            collective_id=0,
            vmem_limit_bytes=40 * 1024 * 1024,
        ),
    )(x_bf, w_bf)
